# Initial kernel scaffold; baseline (speedup 1.0000x reference)
import jax
import jax.numpy as jnp
from jax.experimental import pallas as pl


def kernel(h, x, edges, edge_attr, vec, cfg, Wemb, bemb, TimeEmb, sWe1, sbe1, sWe2, sbe2, sWn1, sbn1, sWn2, sbn2, sWc, aWq, aWk, aWv, theta, Wp):
    raise NotImplementedError("write your pallas kernel here")


# speedup vs baseline: 6.7260x; 6.7260x over previous
import functools

import jax
import jax.numpy as jnp
from jax import lax
from jax.experimental import pallas as pl
from jax.experimental.pallas import tpu as pltpu
from jax.experimental.pallas import tpu_sc as plsc

N = 10000
E = 160000
T = 4
H = 64
F_IN = 128
XW = 8
TW = 128
TN = T * N
TEP = T * E

NC = 2
NS = 16
NW = NC * NS
EPC = 128
NCH_G = TEP // EPC
NCH_S = E // EPC
SPN = 10240
ZR = SPN // NS

def _mesh():
    return plsc.VectorSubcoreMesh(
        core_axis_name="c", subcore_axis_name="s",
        num_cores=NC, num_subcores=NS)

f32 = jnp.float32


def _silu(a):
    return a * jax.nn.sigmoid(a)


def _gather_body(ptab, qtab, rowg, colg, gp, gq,
                 idxr, idxc, bp0, bp1, bq0, bq1, sem0, sem1):
    c = lax.axis_index("c")
    s = lax.axis_index("s")
    wid = s * NC + c
    lo = (wid * NCH_G) // NW
    hi = ((wid + 1) * NCH_G) // NW
    cnt = hi - lo
    lo_c = (lo // 8) * 8
    off = lo - lo_c
    pltpu.sync_copy(rowg.at[pl.ds(lo_c, IDXW_G)], idxr)
    pltpu.sync_copy(colg.at[pl.ds(lo_c, IDXW_G)], idxc)

    bps = (bp0, bp1)
    bqs = (bq0, bq1)
    sems = (sem0, sem1)

    def issue(k, b):
        pltpu.async_copy(ptab.at[idxr.at[off + k]], bps[b], sems[b])
        pltpu.async_copy(qtab.at[idxc.at[off + k]], bqs[b], sems[b])

    issue(0, 0)
    issue(1, 1)

    @pl.loop(0, IDXW_G, step=2)
    def _(k0):
        for b in range(2):
            k = k0 + b

            @pl.when(k < cnt)
            def _():
                pltpu.make_async_copy(
                    ptab.at[pl.ds(0, EPC)], bps[b], sems[b]).wait()
                pltpu.make_async_copy(
                    qtab.at[pl.ds(0, EPC)], bqs[b], sems[b]).wait()
                q = lo + k
                pltpu.sync_copy(bps[b], gp.at[pl.ds(q * EPC, EPC)])
                pltpu.sync_copy(bqs[b], gq.at[pl.ds(q * EPC, EPC)])

                @pl.when(k + 2 < cnt)
                def _():
                    issue(k + 2, b)


IDXW_G = 168
NCH_GP = 5120
NCH_SP = 1280


def _gather_call(ptab, qtab, rowg, colg):
    kfn = pl.kernel(
        _gather_body,
        out_type=[jax.ShapeDtypeStruct((TEP, TW), f32),
                  jax.ShapeDtypeStruct((TEP, TW), f32)],
        mesh=_mesh(),
        scratch_types=[
            pltpu.VMEM((IDXW_G, EPC), jnp.int32),
            pltpu.VMEM((IDXW_G, EPC), jnp.int32),
            pltpu.VMEM((EPC, TW), f32),
            pltpu.VMEM((EPC, TW), f32),
            pltpu.VMEM((EPC, TW), f32),
            pltpu.VMEM((EPC, TW), f32),
            pltpu.SemaphoreType.DMA,
            pltpu.SemaphoreType.DMA,
        ],
    )
    return kfn(ptab, qtab, rowg, colg)


IDXW_S = 48


def _scatter_body(mtr, rows, z128, agg,
                  idxs, nb0, nb1, spm, sem0, sem1):
    c = lax.axis_index("c")
    s = lax.axis_index("s")
    wid = s * NC + c
    lo = (wid * NCH_S) // NW
    hi = ((wid + 1) * NCH_S) // NW
    cnt = hi - lo
    lo_c = (lo // 8) * 8
    off = lo - lo_c
    pltpu.sync_copy(rows.at[pl.ds(lo_c, IDXW_S)], idxs)

    nbs = (nb0, nb1)
    sems = (sem0, sem1)

    for t in range(T):
        pltpu.sync_copy(z128, nb0)
        for d in range(ZR // EPC):
            pltpu.sync_copy(nb0, spm.at[pl.ds(s * ZR + d * EPC, EPC)])
        plsc.subcore_barrier()

        base_row = t * E

        def issue(k, b):
            r = base_row + (lo + k) * EPC
            pltpu.async_copy(mtr.at[pl.ds(r, EPC)], nbs[b], sems[b])

        issue(0, 0)
        issue(1, 1)

        @pl.loop(0, IDXW_S, step=2)
        def _(k0):
            for b in range(2):
                k = k0 + b

                @pl.when(k < cnt)
                def _():
                    pltpu.make_async_copy(
                        mtr.at[pl.ds(0, EPC)], nbs[b], sems[b]).wait()
                    pltpu.sync_copy(nbs[b], spm.at[idxs.at[off + k]], add=True)

                    @pl.when(k + 2 < cnt)
                    def _():
                        issue(k + 2, b)

        plsc.subcore_barrier()
        for d in range(ZR // EPC):
            r = s * ZR + d * EPC
            ro = (c * T + t) * SPN + r
            pltpu.sync_copy(spm.at[pl.ds(r, EPC)], nb0)
            pltpu.sync_copy(nb0, agg.at[pl.ds(ro, EPC)])


def _scatter_call(mtr, rows, z128):
    kfn = pl.kernel(
        _scatter_body,
        out_type=jax.ShapeDtypeStruct((NC * T * SPN, TW), f32),
        mesh=_mesh(),
        scratch_types=[
            pltpu.VMEM((IDXW_S, EPC), jnp.int32),
            pltpu.VMEM((EPC, TW), f32),
            pltpu.VMEM((EPC, TW), f32),
            pltpu.VMEM_SHARED((SPN, TW), f32),
            pltpu.SemaphoreType.DMA,
            pltpu.SemaphoreType.DMA,
        ],
    )
    agg = kfn(mtr, rows, z128)
    return agg.reshape(NC, T, SPN, TW)


BN0 = 1000
BN1 = 2000
BE = 5000


def _embed_body(h_ref, x_ref, wemb_ref, bemb_ref, temb_ref, wab_ref,
                hh_ref, pt_ref, qt_ref):
    h0 = jnp.dot(h_ref[...], wemb_ref[...], preferred_element_type=f32)
    h0 = h0 + bemb_ref[...]
    pq0 = jnp.dot(h0, wab_ref[...], preferred_element_type=f32)
    ttw = jnp.dot(temb_ref[...], wab_ref[...], preferred_element_type=f32)
    z = jnp.zeros((h0.shape[0], TW - H - XW), f32)
    for t in range(T):
        hh_t = h0 + temb_ref[t:t + 1, :]
        hh_ref[t] = hh_t
        pq = pq0 + ttw[t:t + 1, :]
        xt = x_ref[t]
        pt_ref[t] = jnp.concatenate([pq[:, :H], xt, z], axis=-1)
        qt_ref[t] = jnp.concatenate([pq[:, H:], xt, z], axis=-1)


def _embed_call(h, x8, wemb, bemb2, temb, wab):
    return pl.pallas_call(
        _embed_body,
        grid=(N // BN0,),
        in_specs=[
            pl.BlockSpec((BN0, F_IN), lambda i: (i, 0)),
            pl.BlockSpec((T, BN0, XW), lambda i: (0, i, 0)),
            pl.BlockSpec((F_IN, H), lambda i: (0, 0)),
            pl.BlockSpec((1, H), lambda i: (0, 0)),
            pl.BlockSpec((T, H), lambda i: (0, 0)),
            pl.BlockSpec((H, 2 * H), lambda i: (0, 0)),
        ],
        out_specs=[
            pl.BlockSpec((T, BN0, H), lambda i: (0, i, 0)),
            pl.BlockSpec((T, BN0, TW), lambda i: (0, i, 0)),
            pl.BlockSpec((T, BN0, TW), lambda i: (0, i, 0)),
        ],
        out_shape=[
            jax.ShapeDtypeStruct((T, N, H), f32),
            jax.ShapeDtypeStruct((T, N, TW), f32),
            jax.ShapeDtypeStruct((T, N, TW), f32),
        ],
    )(h, x8, wemb, bemb2, temb, wab)


def _edge_body(gp_ref, gq_ref, v_ref, wc_ref, wd_ref, b1_ref, w2_ref, b2_ref,
               scv_ref, mtr_ref):
    gp = gp_ref[...]
    gq = gq_ref[...]
    df = gp - gq
    dfx = df[:, H:H + 16]
    radial = jnp.sum(dfx * dfx, axis=-1, keepdims=True)
    pre = (gp[:, :H] + gq[:, :H]
           + jnp.dot(v_ref[...][:, :H], wc_ref[...],
                     preferred_element_type=f32)
           + radial * wd_ref[...] + b1_ref[...])
    m1 = _silu(pre)
    mm = jnp.dot(m1, w2_ref[...], preferred_element_type=f32) + b2_ref[...]
    mm = _silu(mm)
    sc = jnp.sum(mm * scv_ref[...], axis=-1, keepdims=True)
    trv = df[:, H:H + XW] * sc
    z = jnp.zeros((gp.shape[0], TW - H - XW), f32)
    mtr_ref[...] = jnp.concatenate([mm, trv, z], axis=-1)


def _edge_call(gp, gq, v, wc, wd, b1, w2, b2, scv):
    vw = v.shape[1]
    return pl.pallas_call(
        _edge_body,
        grid=(TEP // BE,),
        in_specs=[
            pl.BlockSpec((BE, TW), lambda i: (i, 0)),
            pl.BlockSpec((BE, TW), lambda i: (i, 0)),
            pl.BlockSpec((BE, vw), lambda i: (i, 0)),
            pl.BlockSpec((H, H), lambda i: (0, 0)),
            pl.BlockSpec((1, H), lambda i: (0, 0)),
            pl.BlockSpec((1, H), lambda i: (0, 0)),
            pl.BlockSpec((H, H), lambda i: (0, 0)),
            pl.BlockSpec((1, H), lambda i: (0, 0)),
            pl.BlockSpec((1, H), lambda i: (0, 0)),
        ],
        out_specs=pl.BlockSpec((BE, TW), lambda i: (i, 0)),
        out_shape=jax.ShapeDtypeStruct((TEP, TW), f32),
    )(gp, gq, v, wc, wd, b1, w2, b2, scv)


def _node_body(hh_ref, x_ref, agg_ref, w1_ref, b1_ref, w2_ref,
               b2_ref, hh2_ref, xx2_ref):
    hb = hh_ref[0]
    acc = agg_ref[0, 0] + agg_ref[1, 0]
    cat = jnp.concatenate([hb, acc[:, :H]], axis=-1)
    mid = _silu(jnp.dot(cat, w1_ref[...], preferred_element_type=f32)
                + b1_ref[...])
    hh2_ref[0] = hb + jnp.dot(mid, w2_ref[...], preferred_element_type=f32) \
        + b2_ref[...]
    xx2_ref[0] = x_ref[0] + acc[:, H:H + XW]


def _node_call(hh, xx, agg, w1, b1, w2, b2):
    return pl.pallas_call(
        _node_body,
        grid=(T, N // BN1),
        in_specs=[
            pl.BlockSpec((1, BN1, H), lambda t, i: (t, i, 0)),
            pl.BlockSpec((1, BN1, XW), lambda t, i: (t, i, 0)),
            pl.BlockSpec((NC, 1, BN1, TW), lambda t, i: (0, t, i, 0)),
            pl.BlockSpec((2 * H, H), lambda t, i: (0, 0)),
            pl.BlockSpec((1, H), lambda t, i: (0, 0)),
            pl.BlockSpec((H, H), lambda t, i: (0, 0)),
            pl.BlockSpec((1, H), lambda t, i: (0, 0)),
        ],
        out_specs=[
            pl.BlockSpec((1, BN1, H), lambda t, i: (t, i, 0)),
            pl.BlockSpec((1, BN1, XW), lambda t, i: (t, i, 0)),
        ],
        out_shape=[
            jax.ShapeDtypeStruct((T, N, H), f32),
            jax.ShapeDtypeStruct((T, N, XW), f32),
        ],
    )(hh, xx, agg, w1, b1, w2, b2)


def _attend(hh_ref, xx_ref, wqkv_ref):
    hb = [hh_ref[t] for t in range(T)]
    qkv = [jnp.dot(hb[t], wqkv_ref[...], preferred_element_type=f32)
           for t in range(T)]
    qs = [a[:, :H] for a in qkv]
    ks = [a[:, H:2 * H] for a in qkv]
    vs = [a[:, 2 * H:] for a in qkv]
    inv = 1.0 / (H ** 0.5)
    hout, xout = [], []
    for t in range(T):
        sc = [jnp.sum(qs[t] * ks[s], axis=-1, keepdims=True) * inv
              for s in range(T)]
        mx = jnp.maximum(jnp.maximum(sc[0], sc[1]), jnp.maximum(sc[2], sc[3]))
        ex = [jnp.exp(c - mx) for c in sc]
        den = ex[0] + ex[1] + ex[2] + ex[3]
        at = [e / den for e in ex]
        hout.append(hb[t] + sum(at[s] * vs[s] for s in range(T)))
        xout.append(sum(at[s] * xx_ref[s] for s in range(T)))
    return hout, xout


def _temporal0_body(hh_ref, xx_ref, wqkv_ref, wab_ref,
                    hh2_ref, xx2_ref, pt_ref, qt_ref):
    hout, xout = _attend(hh_ref, xx_ref, wqkv_ref)
    z = jnp.zeros((hout[0].shape[0], TW - H - XW), f32)
    for t in range(T):
        hh2_ref[t] = hout[t]
        xx2_ref[t] = xout[t]
        pq = jnp.dot(hout[t], wab_ref[...], preferred_element_type=f32)
        pt_ref[t] = jnp.concatenate([pq[:, :H], xout[t], z], axis=-1)
        qt_ref[t] = jnp.concatenate([pq[:, H:], xout[t], z], axis=-1)


def _temporal0_call(hh, xx, wqkv, wab):
    return pl.pallas_call(
        _temporal0_body,
        grid=(N // BN0,),
        in_specs=[
            pl.BlockSpec((T, BN0, H), lambda i: (0, i, 0)),
            pl.BlockSpec((T, BN0, XW), lambda i: (0, i, 0)),
            pl.BlockSpec((H, 3 * H), lambda i: (0, 0)),
            pl.BlockSpec((H, 2 * H), lambda i: (0, 0)),
        ],
        out_specs=[
            pl.BlockSpec((T, BN0, H), lambda i: (0, i, 0)),
            pl.BlockSpec((T, BN0, XW), lambda i: (0, i, 0)),
            pl.BlockSpec((T, BN0, TW), lambda i: (0, i, 0)),
            pl.BlockSpec((T, BN0, TW), lambda i: (0, i, 0)),
        ],
        out_shape=[
            jax.ShapeDtypeStruct((T, N, H), f32),
            jax.ShapeDtypeStruct((T, N, XW), f32),
            jax.ShapeDtypeStruct((T, N, TW), f32),
            jax.ShapeDtypeStruct((T, N, TW), f32),
        ],
    )(hh, xx, wqkv, wab)


def _temporal1_body(hh_ref, xx_ref, wqkv_ref, th_ref, wp_ref, out_ref):
    hout, xout = _attend(hh_ref, xx_ref, wqkv_ref)
    th = th_ref[...]
    xlast = xout[T - 1]
    base = xlast
    for t in range(T):
        base = base + th[0:1, t:t + 1] * (xout[t] - xlast)
    hmean = (hout[0] + hout[1] + hout[2] + hout[3]) * 0.25
    out_ref[...] = base + jnp.dot(hmean, wp_ref[...],
                                  preferred_element_type=f32)


def _temporal1_call(hh, xx, wqkv, theta, wp8):
    return pl.pallas_call(
        _temporal1_body,
        grid=(N // BN0,),
        in_specs=[
            pl.BlockSpec((T, BN0, H), lambda i: (0, i, 0)),
            pl.BlockSpec((T, BN0, XW), lambda i: (0, i, 0)),
            pl.BlockSpec((H, 3 * H), lambda i: (0, 0)),
            pl.BlockSpec((1, T), lambda i: (0, 0)),
            pl.BlockSpec((H, XW), lambda i: (0, 0)),
        ],
        out_specs=pl.BlockSpec((BN0, XW), lambda i: (i, 0)),
        out_shape=jax.ShapeDtypeStruct((N, XW), f32),
    )(hh, xx, wqkv, theta, wp8)


def kernel(h, x, edges, edge_attr, vec, cfg, Wemb, bemb, TimeEmb, sWe1, sbe1,
           sWe2, sbe2, sWn1, sbn1, sWn2, sbn2, sWc, aWq, aWk, aWv, theta, Wp):
    row = edges[0]
    col = edges[1]
    toff = (jnp.arange(T, dtype=jnp.int32) * N)[:, None]
    gpad = NCH_GP * EPC - TEP
    rowg = jnp.pad((row[None, :] + toff).reshape(-1),
                   (0, gpad)).reshape(NCH_GP, EPC)
    colg = jnp.pad((col[None, :] + toff).reshape(-1),
                   (0, gpad)).reshape(NCH_GP, EPC)
    rows = jnp.pad(row, (0, NCH_SP * EPC - E)).reshape(NCH_SP, EPC)

    x8 = jnp.pad(x, ((0, 0), (0, 0), (0, XW - 3)))
    z128 = jnp.zeros((EPC, TW), f32)
    bemb2 = bemb.reshape(1, H)
    wp8 = jnp.zeros((H, XW), f32).at[:, :3].set(Wp)

    wab = [jnp.concatenate([sWe1[li, :H, :], sWe1[li, H:2 * H, :]], axis=1)
           for li in range(2)]
    wc = [sWe1[li, 2 * H:3 * H, :] for li in range(2)]
    wd = [sWe1[li, 3 * H:3 * H + 1, :] for li in range(2)]
    b1 = [sbe1[li].reshape(1, H) for li in range(2)]
    w2 = [sWe2[li] for li in range(2)]
    b2 = [sbe2[li].reshape(1, H) for li in range(2)]
    scv = [sWc[li].reshape(1, H) for li in range(2)]
    wn1 = [sWn1[li] for li in range(2)]
    bn1 = [sbn1[li].reshape(1, H) for li in range(2)]
    wn2 = [sWn2[li] for li in range(2)]
    bn2 = [sbn2[li].reshape(1, H) for li in range(2)]
    wqkv = [jnp.concatenate([aWq[ti], aWk[ti], aWv[ti]], axis=1)
            for ti in range(2)]

    hh0, pt0, qt0 = _embed_call(h, x8, Wemb, bemb2, TimeEmb, wab[0])
    gp0, gq0 = _gather_call(pt0.reshape(TN, TW), qt0.reshape(TN, TW),
                            rowg, colg)
    mtr0 = _edge_call(gp0, gq0, vec.reshape(TEP, H),
                      wc[0], wd[0], b1[0], w2[0], b2[0], scv[0])
    agg0 = _scatter_call(mtr0, rows, z128)
    hh1, xx1 = _node_call(hh0, x8, agg0, wn1[0], bn1[0], wn2[0], bn2[0])
    hh2, xx2, pt1, qt1 = _temporal0_call(hh1, xx1, wqkv[0], wab[1])
    gp1, gq1 = _gather_call(pt1.reshape(TN, TW), qt1.reshape(TN, TW),
                            rowg, colg)
    mtr1 = _edge_call(gp1, gq1, mtr0,
                      wc[1], wd[1], b1[1], w2[1], b2[1], scv[1])
    agg1 = _scatter_call(mtr1, rows, z128)
    hh3, xx3 = _node_call(hh2, xx2, agg1, wn1[1], bn1[1], wn2[1], bn2[1])
    outf = _temporal1_call(hh3, xx3, wqkv[1], theta, wp8)
    return outf[:, :3].reshape(1, N, 3)


# speedup vs baseline: 7.1766x; 1.0670x over previous
import functools

import jax
import jax.numpy as jnp
from jax import lax
from jax.experimental import pallas as pl
from jax.experimental.pallas import tpu as pltpu
from jax.experimental.pallas import tpu_sc as plsc

N = 10000
E = 160000
T = 4
H = 64
F_IN = 128
XW = 8
TW = 128
TN = T * N
TEP = T * E

NC = 2
NS = 16
NW = NC * NS
EPC = 128
NCH_G = TEP // EPC
NCH_S = E // EPC
SPN = 10240
ZR = SPN // NS
IDXW_G = 168
NCH_GP = 5120
NCH_SP = 1280

def _mesh():
    return plsc.VectorSubcoreMesh(
        core_axis_name="c", subcore_axis_name="s",
        num_cores=NC, num_subcores=NS)

f32 = jnp.float32


def _silu(a):
    return a * jax.nn.sigmoid(a)


def _gather_body(ptab, qtab, rowg, colg, g,
                 idxr, idxc, b0, b1, sem0, sem1, sem2, sem3):
    c = lax.axis_index("c")
    s = lax.axis_index("s")
    wid = s * NC + c
    lo = (wid * NCH_G) // NW
    hi = ((wid + 1) * NCH_G) // NW
    cnt = hi - lo
    lo_c = (lo // 8) * 8
    off = lo - lo_c
    pltpu.sync_copy(rowg.at[pl.ds(lo_c, IDXW_G)], idxr)
    pltpu.sync_copy(colg.at[pl.ds(lo_c, IDXW_G)], idxc)

    bufs = (b0, b1)
    semsA = (sem0, sem1)
    semsB = (sem2, sem3)

    def issue_q(k, b):
        pltpu.async_copy(qtab.at[idxc.at[off + k]], bufs[b], semsA[b])

    issue_q(0, 0)
    issue_q(1, 1)

    @pl.loop(0, IDXW_G, step=2)
    def _(k0):
        for b in range(2):
            k = k0 + b

            @pl.when(k < cnt)
            def _():
                pltpu.make_async_copy(
                    qtab.at[pl.ds(0, EPC)], bufs[b], semsA[b]).wait()
                pltpu.async_copy(ptab.at[idxr.at[off + k]], bufs[b],
                                 semsB[b], add=True)
                pltpu.make_async_copy(
                    ptab.at[pl.ds(0, EPC)], bufs[b], semsB[b]).wait()
                q = lo + k
                pltpu.sync_copy(bufs[b], g.at[pl.ds(q * EPC, EPC)])

                @pl.when(k + 2 < cnt)
                def _():
                    issue_q(k + 2, b)


def _gather_call(ptab, qtab, rowg, colg):
    kfn = pl.kernel(
        _gather_body,
        out_type=jax.ShapeDtypeStruct((TEP, TW), f32),
        mesh=_mesh(),
        scratch_types=[
            pltpu.VMEM((IDXW_G, EPC), jnp.int32),
            pltpu.VMEM((IDXW_G, EPC), jnp.int32),
            pltpu.VMEM((EPC, TW), f32),
            pltpu.VMEM((EPC, TW), f32),
            pltpu.SemaphoreType.DMA,
            pltpu.SemaphoreType.DMA,
            pltpu.SemaphoreType.DMA,
            pltpu.SemaphoreType.DMA,
        ],
    )
    return kfn(ptab, qtab, rowg, colg)


IDXW_S = 48


def _scatter_body(mtr, rows, z128, agg,
                  idxs, nb0, nb1, spm, sem0, sem1):
    c = lax.axis_index("c")
    s = lax.axis_index("s")
    wid = s * NC + c
    lo = (wid * NCH_S) // NW
    hi = ((wid + 1) * NCH_S) // NW
    cnt = hi - lo
    lo_c = (lo // 8) * 8
    off = lo - lo_c
    pltpu.sync_copy(rows.at[pl.ds(lo_c, IDXW_S)], idxs)

    nbs = (nb0, nb1)
    sems = (sem0, sem1)

    for t in range(T):
        pltpu.sync_copy(z128, nb0)
        for d in range(ZR // EPC):
            pltpu.sync_copy(nb0, spm.at[pl.ds(s * ZR + d * EPC, EPC)])
        plsc.subcore_barrier()

        base_row = t * E

        def issue(k, b):
            r = base_row + (lo + k) * EPC
            pltpu.async_copy(mtr.at[pl.ds(r, EPC)], nbs[b], sems[b])

        issue(0, 0)
        issue(1, 1)

        @pl.loop(0, IDXW_S, step=2)
        def _(k0):
            for b in range(2):
                k = k0 + b

                @pl.when(k < cnt)
                def _():
                    pltpu.make_async_copy(
                        mtr.at[pl.ds(0, EPC)], nbs[b], sems[b]).wait()
                    pltpu.sync_copy(nbs[b], spm.at[idxs.at[off + k]], add=True)

                    @pl.when(k + 2 < cnt)
                    def _():
                        issue(k + 2, b)

        plsc.subcore_barrier()
        for d in range(ZR // EPC):
            r = s * ZR + d * EPC
            ro = (c * T + t) * SPN + r
            pltpu.sync_copy(spm.at[pl.ds(r, EPC)], nb0)
            pltpu.sync_copy(nb0, agg.at[pl.ds(ro, EPC)])


def _scatter_call(mtr, rows, z128):
    kfn = pl.kernel(
        _scatter_body,
        out_type=jax.ShapeDtypeStruct((NC * T * SPN, TW), f32),
        mesh=_mesh(),
        scratch_types=[
            pltpu.VMEM((IDXW_S, EPC), jnp.int32),
            pltpu.VMEM((EPC, TW), f32),
            pltpu.VMEM((EPC, TW), f32),
            pltpu.VMEM_SHARED((SPN, TW), f32),
            pltpu.SemaphoreType.DMA,
            pltpu.SemaphoreType.DMA,
        ],
    )
    agg = kfn(mtr, rows, z128)
    return agg.reshape(NC, T, SPN, TW)


BN0 = 1000
BN1 = 2000
BE = 5000


def _embed_body(h_ref, x_ref, wemb_ref, bemb_ref, temb_ref, wab_ref,
                hh_ref, pt_ref, qt_ref):
    h0 = jnp.dot(h_ref[...], wemb_ref[...], preferred_element_type=f32)
    h0 = h0 + bemb_ref[...]
    pq0 = jnp.dot(h0, wab_ref[...], preferred_element_type=f32)
    ttw = jnp.dot(temb_ref[...], wab_ref[...], preferred_element_type=f32)
    z = jnp.zeros((h0.shape[0], TW - H - XW), f32)
    for t in range(T):
        hh_t = h0 + temb_ref[t:t + 1, :]
        hh_ref[t] = hh_t
        pq = pq0 + ttw[t:t + 1, :]
        xt = x_ref[t]
        pt_ref[t] = jnp.concatenate([pq[:, :H], xt, z], axis=-1)
        qt_ref[t] = jnp.concatenate([pq[:, H:], -xt, z], axis=-1)


def _embed_call(h, x8, wemb, bemb2, temb, wab):
    return pl.pallas_call(
        _embed_body,
        grid=(N // BN0,),
        in_specs=[
            pl.BlockSpec((BN0, F_IN), lambda i: (i, 0)),
            pl.BlockSpec((T, BN0, XW), lambda i: (0, i, 0)),
            pl.BlockSpec((F_IN, H), lambda i: (0, 0)),
            pl.BlockSpec((1, H), lambda i: (0, 0)),
            pl.BlockSpec((T, H), lambda i: (0, 0)),
            pl.BlockSpec((H, 2 * H), lambda i: (0, 0)),
        ],
        out_specs=[
            pl.BlockSpec((T, BN0, H), lambda i: (0, i, 0)),
            pl.BlockSpec((T, BN0, TW), lambda i: (0, i, 0)),
            pl.BlockSpec((T, BN0, TW), lambda i: (0, i, 0)),
        ],
        out_shape=[
            jax.ShapeDtypeStruct((T, N, H), f32),
            jax.ShapeDtypeStruct((T, N, TW), f32),
            jax.ShapeDtypeStruct((T, N, TW), f32),
        ],
    )(h, x8, wemb, bemb2, temb, wab)


def _edge_body(g_ref, v_ref, wc_ref, wd_ref, b1_ref, w2_ref, b2_ref,
               scv_ref, mtr_ref):
    g = g_ref[...]
    dfx = g[:, H:H + 16]
    radial = jnp.sum(dfx * dfx, axis=-1, keepdims=True)
    pre = (g[:, :H]
           + jnp.dot(v_ref[...][:, :H], wc_ref[...],
                     preferred_element_type=f32)
           + radial * wd_ref[...] + b1_ref[...])
    m1 = _silu(pre)
    mm = jnp.dot(m1, w2_ref[...], preferred_element_type=f32) + b2_ref[...]
    mm = _silu(mm)
    sc = jnp.sum(mm * scv_ref[...], axis=-1, keepdims=True)
    trv = g[:, H:H + XW] * sc
    z = jnp.zeros((g.shape[0], TW - H - XW), f32)
    mtr_ref[...] = jnp.concatenate([mm, trv, z], axis=-1)


def _edge_call(g, v, wc, wd, b1, w2, b2, scv):
    vw = v.shape[1]
    return pl.pallas_call(
        _edge_body,
        grid=(TEP // BE,),
        in_specs=[
            pl.BlockSpec((BE, TW), lambda i: (i, 0)),
            pl.BlockSpec((BE, vw), lambda i: (i, 0)),
            pl.BlockSpec((H, H), lambda i: (0, 0)),
            pl.BlockSpec((1, H), lambda i: (0, 0)),
            pl.BlockSpec((1, H), lambda i: (0, 0)),
            pl.BlockSpec((H, H), lambda i: (0, 0)),
            pl.BlockSpec((1, H), lambda i: (0, 0)),
            pl.BlockSpec((1, H), lambda i: (0, 0)),
        ],
        out_specs=pl.BlockSpec((BE, TW), lambda i: (i, 0)),
        out_shape=jax.ShapeDtypeStruct((TEP, TW), f32),
    )(g, v, wc, wd, b1, w2, b2, scv)


def _node_body(hh_ref, x_ref, agg_ref, w1_ref, b1_ref, w2_ref,
               b2_ref, hh2_ref, xx2_ref):
    hb = hh_ref[0]
    acc = agg_ref[0, 0] + agg_ref[1, 0]
    cat = jnp.concatenate([hb, acc[:, :H]], axis=-1)
    mid = _silu(jnp.dot(cat, w1_ref[...], preferred_element_type=f32)
                + b1_ref[...])
    hh2_ref[0] = hb + jnp.dot(mid, w2_ref[...], preferred_element_type=f32) \
        + b2_ref[...]
    xx2_ref[0] = x_ref[0] + acc[:, H:H + XW]


def _node_call(hh, xx, agg, w1, b1, w2, b2):
    return pl.pallas_call(
        _node_body,
        grid=(T, N // BN1),
        in_specs=[
            pl.BlockSpec((1, BN1, H), lambda t, i: (t, i, 0)),
            pl.BlockSpec((1, BN1, XW), lambda t, i: (t, i, 0)),
            pl.BlockSpec((NC, 1, BN1, TW), lambda t, i: (0, t, i, 0)),
            pl.BlockSpec((2 * H, H), lambda t, i: (0, 0)),
            pl.BlockSpec((1, H), lambda t, i: (0, 0)),
            pl.BlockSpec((H, H), lambda t, i: (0, 0)),
            pl.BlockSpec((1, H), lambda t, i: (0, 0)),
        ],
        out_specs=[
            pl.BlockSpec((1, BN1, H), lambda t, i: (t, i, 0)),
            pl.BlockSpec((1, BN1, XW), lambda t, i: (t, i, 0)),
        ],
        out_shape=[
            jax.ShapeDtypeStruct((T, N, H), f32),
            jax.ShapeDtypeStruct((T, N, XW), f32),
        ],
    )(hh, xx, agg, w1, b1, w2, b2)


def _attend(hh_ref, xx_ref, wqkv_ref):
    hb = [hh_ref[t] for t in range(T)]
    qkv = [jnp.dot(hb[t], wqkv_ref[...], preferred_element_type=f32)
           for t in range(T)]
    qs = [a[:, :H] for a in qkv]
    ks = [a[:, H:2 * H] for a in qkv]
    vs = [a[:, 2 * H:] for a in qkv]
    inv = 1.0 / (H ** 0.5)
    hout, xout = [], []
    for t in range(T):
        sc = [jnp.sum(qs[t] * ks[s], axis=-1, keepdims=True) * inv
              for s in range(T)]
        mx = jnp.maximum(jnp.maximum(sc[0], sc[1]), jnp.maximum(sc[2], sc[3]))
        ex = [jnp.exp(c - mx) for c in sc]
        den = ex[0] + ex[1] + ex[2] + ex[3]
        at = [e / den for e in ex]
        hout.append(hb[t] + sum(at[s] * vs[s] for s in range(T)))
        xout.append(sum(at[s] * xx_ref[s] for s in range(T)))
    return hout, xout


def _temporal0_body(hh_ref, xx_ref, wqkv_ref, wab_ref,
                    hh2_ref, xx2_ref, pt_ref, qt_ref):
    hout, xout = _attend(hh_ref, xx_ref, wqkv_ref)
    z = jnp.zeros((hout[0].shape[0], TW - H - XW), f32)
    for t in range(T):
        hh2_ref[t] = hout[t]
        xx2_ref[t] = xout[t]
        pq = jnp.dot(hout[t], wab_ref[...], preferred_element_type=f32)
        pt_ref[t] = jnp.concatenate([pq[:, :H], xout[t], z], axis=-1)
        qt_ref[t] = jnp.concatenate([pq[:, H:], -xout[t], z], axis=-1)


def _temporal0_call(hh, xx, wqkv, wab):
    return pl.pallas_call(
        _temporal0_body,
        grid=(N // BN0,),
        in_specs=[
            pl.BlockSpec((T, BN0, H), lambda i: (0, i, 0)),
            pl.BlockSpec((T, BN0, XW), lambda i: (0, i, 0)),
            pl.BlockSpec((H, 3 * H), lambda i: (0, 0)),
            pl.BlockSpec((H, 2 * H), lambda i: (0, 0)),
        ],
        out_specs=[
            pl.BlockSpec((T, BN0, H), lambda i: (0, i, 0)),
            pl.BlockSpec((T, BN0, XW), lambda i: (0, i, 0)),
            pl.BlockSpec((T, BN0, TW), lambda i: (0, i, 0)),
            pl.BlockSpec((T, BN0, TW), lambda i: (0, i, 0)),
        ],
        out_shape=[
            jax.ShapeDtypeStruct((T, N, H), f32),
            jax.ShapeDtypeStruct((T, N, XW), f32),
            jax.ShapeDtypeStruct((T, N, TW), f32),
            jax.ShapeDtypeStruct((T, N, TW), f32),
        ],
    )(hh, xx, wqkv, wab)


def _temporal1_body(hh_ref, xx_ref, wqkv_ref, th_ref, wp_ref, out_ref):
    hout, xout = _attend(hh_ref, xx_ref, wqkv_ref)
    th = th_ref[...]
    xlast = xout[T - 1]
    base = xlast
    for t in range(T):
        base = base + th[0:1, t:t + 1] * (xout[t] - xlast)
    hmean = (hout[0] + hout[1] + hout[2] + hout[3]) * 0.25
    out_ref[...] = base + jnp.dot(hmean, wp_ref[...],
                                  preferred_element_type=f32)


def _temporal1_call(hh, xx, wqkv, theta, wp8):
    return pl.pallas_call(
        _temporal1_body,
        grid=(N // BN0,),
        in_specs=[
            pl.BlockSpec((T, BN0, H), lambda i: (0, i, 0)),
            pl.BlockSpec((T, BN0, XW), lambda i: (0, i, 0)),
            pl.BlockSpec((H, 3 * H), lambda i: (0, 0)),
            pl.BlockSpec((1, T), lambda i: (0, 0)),
            pl.BlockSpec((H, XW), lambda i: (0, 0)),
        ],
        out_specs=pl.BlockSpec((BN0, XW), lambda i: (i, 0)),
        out_shape=jax.ShapeDtypeStruct((N, XW), f32),
    )(hh, xx, wqkv, theta, wp8)


def kernel(h, x, edges, edge_attr, vec, cfg, Wemb, bemb, TimeEmb, sWe1, sbe1,
           sWe2, sbe2, sWn1, sbn1, sWn2, sbn2, sWc, aWq, aWk, aWv, theta, Wp):
    row = edges[0]
    col = edges[1]
    toff = (jnp.arange(T, dtype=jnp.int32) * N)[:, None]
    gpad = NCH_GP * EPC - TEP
    rowg = jnp.pad((row[None, :] + toff).reshape(-1),
                   (0, gpad)).reshape(NCH_GP, EPC)
    colg = jnp.pad((col[None, :] + toff).reshape(-1),
                   (0, gpad)).reshape(NCH_GP, EPC)
    rows = jnp.pad(row, (0, NCH_SP * EPC - E)).reshape(NCH_SP, EPC)

    x8 = jnp.pad(x, ((0, 0), (0, 0), (0, XW - 3)))
    z128 = jnp.zeros((EPC, TW), f32)
    bemb2 = bemb.reshape(1, H)
    wp8 = jnp.zeros((H, XW), f32).at[:, :3].set(Wp)

    wab = [jnp.concatenate([sWe1[li, :H, :], sWe1[li, H:2 * H, :]], axis=1)
           for li in range(2)]
    wc = [sWe1[li, 2 * H:3 * H, :] for li in range(2)]
    wd = [sWe1[li, 3 * H:3 * H + 1, :] for li in range(2)]
    b1 = [sbe1[li].reshape(1, H) for li in range(2)]
    w2 = [sWe2[li] for li in range(2)]
    b2 = [sbe2[li].reshape(1, H) for li in range(2)]
    scv = [sWc[li].reshape(1, H) for li in range(2)]
    wn1 = [sWn1[li] for li in range(2)]
    bn1 = [sbn1[li].reshape(1, H) for li in range(2)]
    wn2 = [sWn2[li] for li in range(2)]
    bn2 = [sbn2[li].reshape(1, H) for li in range(2)]
    wqkv = [jnp.concatenate([aWq[ti], aWk[ti], aWv[ti]], axis=1)
            for ti in range(2)]

    hh0, pt0, qt0 = _embed_call(h, x8, Wemb, bemb2, TimeEmb, wab[0])
    g0 = _gather_call(pt0.reshape(TN, TW), qt0.reshape(TN, TW),
                      rowg, colg)
    mtr0 = _edge_call(g0, vec.reshape(TEP, H),
                      wc[0], wd[0], b1[0], w2[0], b2[0], scv[0])
    agg0 = _scatter_call(mtr0, rows, z128)
    hh1, xx1 = _node_call(hh0, x8, agg0, wn1[0], bn1[0], wn2[0], bn2[0])
    hh2, xx2, pt1, qt1 = _temporal0_call(hh1, xx1, wqkv[0], wab[1])
    g1 = _gather_call(pt1.reshape(TN, TW), qt1.reshape(TN, TW),
                      rowg, colg)
    mtr1 = _edge_call(g1, mtr0,
                      wc[1], wd[1], b1[1], w2[1], b2[1], scv[1])
    agg1 = _scatter_call(mtr1, rows, z128)
    hh3, xx3 = _node_call(hh2, xx2, agg1, wn1[1], bn1[1], wn2[1], bn2[1])
    outf = _temporal1_call(hh3, xx3, wqkv[1], theta, wp8)
    return outf[:, :3].reshape(1, N, 3)


# speedup vs baseline: 7.6882x; 1.0713x over previous
import functools

import jax
import jax.numpy as jnp
from jax import lax
from jax.experimental import pallas as pl
from jax.experimental.pallas import tpu as pltpu
from jax.experimental.pallas import tpu_sc as plsc

N = 10000
E = 160000
T = 4
H = 64
F_IN = 128
XW = 8
TW = 128
TN = T * N
TEP = T * E

NC = 2
NS = 16
NW = NC * NS
EPC = 128
NCH_G = TEP // EPC
NCH_S = E // EPC
SPN = 10240
ZR = SPN // NS
IDXW_G = 168
NCH_GP = 5120
NCH_SP = 1280

def _mesh():
    return plsc.VectorSubcoreMesh(
        core_axis_name="c", subcore_axis_name="s",
        num_cores=NC, num_subcores=NS)

f32 = jnp.float32


def _silu(a):
    return a * jax.nn.sigmoid(a)


def _gather_body(ptab, qtab, rowg, colg, g,
                 idxr, idxc, b0, b1, b2, b3,
                 sa0, sa1, sa2, sa3, sb0, sb1, sb2, sb3):
    c = lax.axis_index("c")
    s = lax.axis_index("s")
    wid = s * NC + c
    lo = (wid * NCH_G) // NW
    hi = ((wid + 1) * NCH_G) // NW
    cnt = hi - lo
    lo_c = (lo // 8) * 8
    off = lo - lo_c
    pltpu.sync_copy(rowg.at[pl.ds(lo_c, IDXW_G)], idxr)
    pltpu.sync_copy(colg.at[pl.ds(lo_c, IDXW_G)], idxc)

    bufs = (b0, b1, b2, b3)
    semsA = (sa0, sa1, sa2, sa3)
    semsB = (sb0, sb1, sb2, sb3)

    def issue_q(k, b):
        pltpu.async_copy(qtab.at[idxc.at[off + k]], bufs[b], semsA[b])

    for b in range(4):
        issue_q(b, b)

    @pl.loop(0, IDXW_G, step=4)
    def _(k0):
        for b in range(4):
            k = k0 + b

            @pl.when(k < cnt)
            def _():
                pltpu.make_async_copy(
                    qtab.at[pl.ds(0, EPC)], bufs[b], semsA[b]).wait()
                pltpu.async_copy(ptab.at[idxr.at[off + k]], bufs[b],
                                 semsB[b], add=True)

        for b in range(4):
            k = k0 + b

            @pl.when(k < cnt)
            def _():
                pltpu.make_async_copy(
                    ptab.at[pl.ds(0, EPC)], bufs[b], semsB[b]).wait()
                q = lo + k
                pltpu.sync_copy(bufs[b], g.at[pl.ds(q * EPC, EPC)])

                @pl.when(k + 4 < cnt)
                def _():
                    issue_q(k + 4, b)


def _gather_call(ptab, qtab, rowg, colg):
    kfn = pl.kernel(
        _gather_body,
        out_type=jax.ShapeDtypeStruct((TEP, TW), f32),
        mesh=_mesh(),
        scratch_types=(
            [pltpu.VMEM((IDXW_G, EPC), jnp.int32)] * 2
            + [pltpu.VMEM((EPC, TW), f32)] * 4
            + [pltpu.SemaphoreType.DMA] * 8
        ),
    )
    return kfn(ptab, qtab, rowg, colg)


IDXW_S = 48


def _scatter_body(mtr, rows, z128, agg,
                  idxs, nb0, nb1, spm, sem0, sem1):
    c = lax.axis_index("c")
    s = lax.axis_index("s")
    wid = s * NC + c
    lo = (wid * NCH_S) // NW
    hi = ((wid + 1) * NCH_S) // NW
    cnt = hi - lo
    lo_c = (lo // 8) * 8
    off = lo - lo_c
    pltpu.sync_copy(rows.at[pl.ds(lo_c, IDXW_S)], idxs)

    nbs = (nb0, nb1)
    sems = (sem0, sem1)

    for t in range(T):
        pltpu.sync_copy(z128, nb0)
        for d in range(ZR // EPC):
            pltpu.sync_copy(nb0, spm.at[pl.ds(s * ZR + d * EPC, EPC)])
        plsc.subcore_barrier()

        base_row = t * E

        def issue(k, b):
            r = base_row + (lo + k) * EPC
            pltpu.async_copy(mtr.at[pl.ds(r, EPC)], nbs[b], sems[b])

        issue(0, 0)
        issue(1, 1)

        @pl.loop(0, IDXW_S, step=2)
        def _(k0):
            for b in range(2):
                k = k0 + b

                @pl.when(k < cnt)
                def _():
                    pltpu.make_async_copy(
                        mtr.at[pl.ds(0, EPC)], nbs[b], sems[b]).wait()
                    pltpu.sync_copy(nbs[b], spm.at[idxs.at[off + k]], add=True)

                    @pl.when(k + 2 < cnt)
                    def _():
                        issue(k + 2, b)

        plsc.subcore_barrier()
        for d in range(ZR // EPC):
            r = s * ZR + d * EPC
            ro = (c * T + t) * SPN + r
            pltpu.sync_copy(spm.at[pl.ds(r, EPC)], nb0)
            pltpu.sync_copy(nb0, agg.at[pl.ds(ro, EPC)])


def _scatter_call(mtr, rows, z128):
    kfn = pl.kernel(
        _scatter_body,
        out_type=jax.ShapeDtypeStruct((NC * T * SPN, TW), f32),
        mesh=_mesh(),
        scratch_types=[
            pltpu.VMEM((IDXW_S, EPC), jnp.int32),
            pltpu.VMEM((EPC, TW), f32),
            pltpu.VMEM((EPC, TW), f32),
            pltpu.VMEM_SHARED((SPN, TW), f32),
            pltpu.SemaphoreType.DMA,
            pltpu.SemaphoreType.DMA,
        ],
    )
    agg = kfn(mtr, rows, z128)
    return agg.reshape(NC, T, SPN, TW)


BN0 = 1000
BN1 = 2000
BE = 5000


def _embed_body(h_ref, x_ref, wemb_ref, bemb_ref, temb_ref, wab_ref,
                hh_ref, pt_ref, qt_ref):
    h0 = jnp.dot(h_ref[...], wemb_ref[...], preferred_element_type=f32)
    h0 = h0 + bemb_ref[...]
    pq0 = jnp.dot(h0, wab_ref[...], preferred_element_type=f32)
    ttw = jnp.dot(temb_ref[...], wab_ref[...], preferred_element_type=f32)
    z = jnp.zeros((h0.shape[0], TW - H - XW), f32)
    for t in range(T):
        hh_t = h0 + temb_ref[t:t + 1, :]
        hh_ref[t] = hh_t
        pq = pq0 + ttw[t:t + 1, :]
        xt = x_ref[t]
        pt_ref[t] = jnp.concatenate([pq[:, :H], xt, z], axis=-1)
        qt_ref[t] = jnp.concatenate([pq[:, H:], -xt, z], axis=-1)


def _embed_call(h, x8, wemb, bemb2, temb, wab):
    return pl.pallas_call(
        _embed_body,
        grid=(N // BN0,),
        in_specs=[
            pl.BlockSpec((BN0, F_IN), lambda i: (i, 0)),
            pl.BlockSpec((T, BN0, XW), lambda i: (0, i, 0)),
            pl.BlockSpec((F_IN, H), lambda i: (0, 0)),
            pl.BlockSpec((1, H), lambda i: (0, 0)),
            pl.BlockSpec((T, H), lambda i: (0, 0)),
            pl.BlockSpec((H, 2 * H), lambda i: (0, 0)),
        ],
        out_specs=[
            pl.BlockSpec((T, BN0, H), lambda i: (0, i, 0)),
            pl.BlockSpec((T, BN0, TW), lambda i: (0, i, 0)),
            pl.BlockSpec((T, BN0, TW), lambda i: (0, i, 0)),
        ],
        out_shape=[
            jax.ShapeDtypeStruct((T, N, H), f32),
            jax.ShapeDtypeStruct((T, N, TW), f32),
            jax.ShapeDtypeStruct((T, N, TW), f32),
        ],
    )(h, x8, wemb, bemb2, temb, wab)


def _edge_body(g_ref, v_ref, wc_ref, wd_ref, b1_ref, w2_ref, b2_ref,
               scv_ref, mtr_ref):
    g = g_ref[...]
    dfx = g[:, H:H + 16]
    radial = jnp.sum(dfx * dfx, axis=-1, keepdims=True)
    pre = (g[:, :H]
           + jnp.dot(v_ref[...][:, :H], wc_ref[...],
                     preferred_element_type=f32)
           + radial * wd_ref[...] + b1_ref[...])
    m1 = _silu(pre)
    mm = jnp.dot(m1, w2_ref[...], preferred_element_type=f32) + b2_ref[...]
    mm = _silu(mm)
    sc = jnp.sum(mm * scv_ref[...], axis=-1, keepdims=True)
    trv = g[:, H:H + XW] * sc
    z = jnp.zeros((g.shape[0], TW - H - XW), f32)
    mtr_ref[...] = jnp.concatenate([mm, trv, z], axis=-1)


def _edge_call(g, v, wc, wd, b1, w2, b2, scv):
    vw = v.shape[1]
    return pl.pallas_call(
        _edge_body,
        grid=(TEP // BE,),
        in_specs=[
            pl.BlockSpec((BE, TW), lambda i: (i, 0)),
            pl.BlockSpec((BE, vw), lambda i: (i, 0)),
            pl.BlockSpec((H, H), lambda i: (0, 0)),
            pl.BlockSpec((1, H), lambda i: (0, 0)),
            pl.BlockSpec((1, H), lambda i: (0, 0)),
            pl.BlockSpec((H, H), lambda i: (0, 0)),
            pl.BlockSpec((1, H), lambda i: (0, 0)),
            pl.BlockSpec((1, H), lambda i: (0, 0)),
        ],
        out_specs=pl.BlockSpec((BE, TW), lambda i: (i, 0)),
        out_shape=jax.ShapeDtypeStruct((TEP, TW), f32),
    )(g, v, wc, wd, b1, w2, b2, scv)


def _node_body(hh_ref, x_ref, agg_ref, w1_ref, b1_ref, w2_ref,
               b2_ref, hh2_ref, xx2_ref):
    hb = hh_ref[0]
    acc = agg_ref[0, 0] + agg_ref[1, 0]
    cat = jnp.concatenate([hb, acc[:, :H]], axis=-1)
    mid = _silu(jnp.dot(cat, w1_ref[...], preferred_element_type=f32)
                + b1_ref[...])
    hh2_ref[0] = hb + jnp.dot(mid, w2_ref[...], preferred_element_type=f32) \
        + b2_ref[...]
    xx2_ref[0] = x_ref[0] + acc[:, H:H + XW]


def _node_call(hh, xx, agg, w1, b1, w2, b2):
    return pl.pallas_call(
        _node_body,
        grid=(T, N // BN1),
        in_specs=[
            pl.BlockSpec((1, BN1, H), lambda t, i: (t, i, 0)),
            pl.BlockSpec((1, BN1, XW), lambda t, i: (t, i, 0)),
            pl.BlockSpec((NC, 1, BN1, TW), lambda t, i: (0, t, i, 0)),
            pl.BlockSpec((2 * H, H), lambda t, i: (0, 0)),
            pl.BlockSpec((1, H), lambda t, i: (0, 0)),
            pl.BlockSpec((H, H), lambda t, i: (0, 0)),
            pl.BlockSpec((1, H), lambda t, i: (0, 0)),
        ],
        out_specs=[
            pl.BlockSpec((1, BN1, H), lambda t, i: (t, i, 0)),
            pl.BlockSpec((1, BN1, XW), lambda t, i: (t, i, 0)),
        ],
        out_shape=[
            jax.ShapeDtypeStruct((T, N, H), f32),
            jax.ShapeDtypeStruct((T, N, XW), f32),
        ],
    )(hh, xx, agg, w1, b1, w2, b2)


def _attend(hh_ref, xx_ref, wqkv_ref):
    hb = [hh_ref[t] for t in range(T)]
    qkv = [jnp.dot(hb[t], wqkv_ref[...], preferred_element_type=f32)
           for t in range(T)]
    qs = [a[:, :H] for a in qkv]
    ks = [a[:, H:2 * H] for a in qkv]
    vs = [a[:, 2 * H:] for a in qkv]
    inv = 1.0 / (H ** 0.5)
    hout, xout = [], []
    for t in range(T):
        sc = [jnp.sum(qs[t] * ks[s], axis=-1, keepdims=True) * inv
              for s in range(T)]
        mx = jnp.maximum(jnp.maximum(sc[0], sc[1]), jnp.maximum(sc[2], sc[3]))
        ex = [jnp.exp(c - mx) for c in sc]
        den = ex[0] + ex[1] + ex[2] + ex[3]
        at = [e / den for e in ex]
        hout.append(hb[t] + sum(at[s] * vs[s] for s in range(T)))
        xout.append(sum(at[s] * xx_ref[s] for s in range(T)))
    return hout, xout


def _temporal0_body(hh_ref, xx_ref, wqkv_ref, wab_ref,
                    hh2_ref, xx2_ref, pt_ref, qt_ref):
    hout, xout = _attend(hh_ref, xx_ref, wqkv_ref)
    z = jnp.zeros((hout[0].shape[0], TW - H - XW), f32)
    for t in range(T):
        hh2_ref[t] = hout[t]
        xx2_ref[t] = xout[t]
        pq = jnp.dot(hout[t], wab_ref[...], preferred_element_type=f32)
        pt_ref[t] = jnp.concatenate([pq[:, :H], xout[t], z], axis=-1)
        qt_ref[t] = jnp.concatenate([pq[:, H:], -xout[t], z], axis=-1)


def _temporal0_call(hh, xx, wqkv, wab):
    return pl.pallas_call(
        _temporal0_body,
        grid=(N // BN0,),
        in_specs=[
            pl.BlockSpec((T, BN0, H), lambda i: (0, i, 0)),
            pl.BlockSpec((T, BN0, XW), lambda i: (0, i, 0)),
            pl.BlockSpec((H, 3 * H), lambda i: (0, 0)),
            pl.BlockSpec((H, 2 * H), lambda i: (0, 0)),
        ],
        out_specs=[
            pl.BlockSpec((T, BN0, H), lambda i: (0, i, 0)),
            pl.BlockSpec((T, BN0, XW), lambda i: (0, i, 0)),
            pl.BlockSpec((T, BN0, TW), lambda i: (0, i, 0)),
            pl.BlockSpec((T, BN0, TW), lambda i: (0, i, 0)),
        ],
        out_shape=[
            jax.ShapeDtypeStruct((T, N, H), f32),
            jax.ShapeDtypeStruct((T, N, XW), f32),
            jax.ShapeDtypeStruct((T, N, TW), f32),
            jax.ShapeDtypeStruct((T, N, TW), f32),
        ],
    )(hh, xx, wqkv, wab)


def _temporal1_body(hh_ref, xx_ref, wqkv_ref, th_ref, wp_ref, out_ref):
    hout, xout = _attend(hh_ref, xx_ref, wqkv_ref)
    th = th_ref[...]
    xlast = xout[T - 1]
    base = xlast
    for t in range(T):
        base = base + th[0:1, t:t + 1] * (xout[t] - xlast)
    hmean = (hout[0] + hout[1] + hout[2] + hout[3]) * 0.25
    out_ref[...] = base + jnp.dot(hmean, wp_ref[...],
                                  preferred_element_type=f32)


def _temporal1_call(hh, xx, wqkv, theta, wp8):
    return pl.pallas_call(
        _temporal1_body,
        grid=(N // BN0,),
        in_specs=[
            pl.BlockSpec((T, BN0, H), lambda i: (0, i, 0)),
            pl.BlockSpec((T, BN0, XW), lambda i: (0, i, 0)),
            pl.BlockSpec((H, 3 * H), lambda i: (0, 0)),
            pl.BlockSpec((1, T), lambda i: (0, 0)),
            pl.BlockSpec((H, XW), lambda i: (0, 0)),
        ],
        out_specs=pl.BlockSpec((BN0, XW), lambda i: (i, 0)),
        out_shape=jax.ShapeDtypeStruct((N, XW), f32),
    )(hh, xx, wqkv, theta, wp8)


def kernel(h, x, edges, edge_attr, vec, cfg, Wemb, bemb, TimeEmb, sWe1, sbe1,
           sWe2, sbe2, sWn1, sbn1, sWn2, sbn2, sWc, aWq, aWk, aWv, theta, Wp):
    row = edges[0]
    col = edges[1]
    toff = (jnp.arange(T, dtype=jnp.int32) * N)[:, None]
    gpad = NCH_GP * EPC - TEP
    rowg = jnp.pad((row[None, :] + toff).reshape(-1),
                   (0, gpad)).reshape(NCH_GP, EPC)
    colg = jnp.pad((col[None, :] + toff).reshape(-1),
                   (0, gpad)).reshape(NCH_GP, EPC)
    rows = jnp.pad(row, (0, NCH_SP * EPC - E)).reshape(NCH_SP, EPC)

    x8 = jnp.pad(x, ((0, 0), (0, 0), (0, XW - 3)))
    z128 = jnp.zeros((EPC, TW), f32)
    bemb2 = bemb.reshape(1, H)
    wp8 = jnp.zeros((H, XW), f32).at[:, :3].set(Wp)

    wab = [jnp.concatenate([sWe1[li, :H, :], sWe1[li, H:2 * H, :]], axis=1)
           for li in range(2)]
    wc = [sWe1[li, 2 * H:3 * H, :] for li in range(2)]
    wd = [sWe1[li, 3 * H:3 * H + 1, :] for li in range(2)]
    b1 = [sbe1[li].reshape(1, H) for li in range(2)]
    w2 = [sWe2[li] for li in range(2)]
    b2 = [sbe2[li].reshape(1, H) for li in range(2)]
    scv = [sWc[li].reshape(1, H) for li in range(2)]
    wn1 = [sWn1[li] for li in range(2)]
    bn1 = [sbn1[li].reshape(1, H) for li in range(2)]
    wn2 = [sWn2[li] for li in range(2)]
    bn2 = [sbn2[li].reshape(1, H) for li in range(2)]
    wqkv = [jnp.concatenate([aWq[ti], aWk[ti], aWv[ti]], axis=1)
            for ti in range(2)]

    hh0, pt0, qt0 = _embed_call(h, x8, Wemb, bemb2, TimeEmb, wab[0])
    g0 = _gather_call(pt0.reshape(TN, TW), qt0.reshape(TN, TW),
                      rowg, colg)
    mtr0 = _edge_call(g0, vec.reshape(TEP, H),
                      wc[0], wd[0], b1[0], w2[0], b2[0], scv[0])
    agg0 = _scatter_call(mtr0, rows, z128)
    hh1, xx1 = _node_call(hh0, x8, agg0, wn1[0], bn1[0], wn2[0], bn2[0])
    hh2, xx2, pt1, qt1 = _temporal0_call(hh1, xx1, wqkv[0], wab[1])
    g1 = _gather_call(pt1.reshape(TN, TW), qt1.reshape(TN, TW),
                      rowg, colg)
    mtr1 = _edge_call(g1, mtr0,
                      wc[1], wd[1], b1[1], w2[1], b2[1], scv[1])
    agg1 = _scatter_call(mtr1, rows, z128)
    hh3, xx3 = _node_call(hh2, xx2, agg1, wn1[1], bn1[1], wn2[1], bn2[1])
    outf = _temporal1_call(hh3, xx3, wqkv[1], theta, wp8)
    return outf[:, :3].reshape(1, N, 3)
